# Initial kernel scaffold; baseline (speedup 1.0000x reference)
import jax
import jax.numpy as jnp
from jax.experimental import pallas as pl


def kernel(roleset, properties, thematic_roles, roleset_table, W_rl, b_rl, value_table, W_t, b_t, ln_gamma, ln_beta, pos_role, pos_obj, mark_absent):
    raise NotImplementedError("write your pallas kernel here")


# speedup vs baseline: 47.7080x; 47.7080x over previous
import functools

import jax
import jax.numpy as jnp
from jax import lax
from jax.experimental import pallas as pl
from jax.experimental.pallas import tpu as pltpu
from jax.experimental.pallas import tpu_sc as plsc

_BS = 16384
_R = 8
_P = 18
_D = 64
_NV = 4
_ROWS = _BS * _R
_BLK = 2048
_EPS = 1e-5


def _prep_body(rt_ref, wrl_ref, brl_ref, posr_ref, vt_ref, wt_ref,
               r2_ref, c_ref):
    r2_ref[...] = (
        jnp.dot(rt_ref[...], wrl_ref[...], preferred_element_type=jnp.float32)
        + brl_ref[...] + posr_ref[...]
    )
    for p in range(_P):
        c_ref[pl.ds(_NV * p, _NV), :] = jnp.dot(
            vt_ref[pl.ds(_NV * p, _NV), :],
            wt_ref[pl.ds(_D * p, _D), :],
            preferred_element_type=jnp.float32,
        )


def _obj_body(props_ref, tr_ref, cw_ref, bt_ref, g_ref, b_ref, mark_ref,
              pos_ref, out_ref):
    props = props_ref[...]
    h = bt_ref[...]
    for w in range(_NV):
        oh = (props == w).astype(jnp.float32)
        h = h + jnp.dot(oh, cw_ref[w], preferred_element_type=jnp.float32)
    h = jnp.maximum(h, 0.0)
    mu = jnp.mean(h, axis=1, keepdims=True)
    var = jnp.mean((h - mu) * (h - mu), axis=1, keepdims=True)
    y = (h - mu) / jnp.sqrt(var + _EPS) * g_ref[...] + b_ref[...]
    pad = tr_ref[...] == -1
    out_ref[...] = jnp.where(pad, mark_ref[...], y) + pos_ref[...]


def kernel(roleset, properties, thematic_roles, roleset_table, W_rl, b_rl,
           value_table, W_t, b_t, ln_gamma, ln_beta, pos_role, pos_obj,
           mark_absent):
    f32 = jnp.float32
    roleset = roleset.astype(jnp.int32)
    props2 = properties.reshape(_ROWS, _P).astype(jnp.int32)
    tr2 = thematic_roles.reshape(_ROWS, 1).astype(jnp.int32)

    n_rs = roleset_table.shape[0]
    r2, c_tab = pl.pallas_call(
        _prep_body,
        out_shape=[
            jax.ShapeDtypeStruct((n_rs, _D), f32),
            jax.ShapeDtypeStruct((_NV * _P, _D), f32),
        ],
    )(roleset_table, W_rl, b_rl.reshape(1, _D), pos_role.reshape(1, _D),
      value_table, W_t)

    cw = c_tab.reshape(_P, _NV, _D).transpose(1, 0, 2)

    info = plsc.get_sparse_core_info()
    nc, ns = info.num_cores, info.num_subcores
    nw = nc * ns
    per_w = _BS // nw
    ch = 128
    n_ch = per_w // ch

    mesh = plsc.VectorSubcoreMesh(core_axis_name="c", subcore_axis_name="s")

    @functools.partial(
        pl.kernel,
        out_type=jax.ShapeDtypeStruct((_BS, _D), f32),
        mesh=mesh,
        scratch_types=[
            pltpu.VMEM((ch,), jnp.int32),
            pltpu.VMEM((per_w, _D), f32),
            pltpu.SemaphoreType.DMA,
        ],
        compiler_params=pltpu.CompilerParams(use_tc_tiling_on_sc=False),
    )
    def _role_gather(idx_hbm, tab_hbm, out_hbm, idx_v, rows_v, sem):
        wid = lax.axis_index("s") * nc + lax.axis_index("c")
        base = wid * per_w
        for j in range(n_ch):
            pltpu.sync_copy(idx_hbm.at[pl.ds(base + j * ch, ch)], idx_v)
            pltpu.async_copy(
                tab_hbm.at[idx_v], rows_v.at[pl.ds(j * ch, ch)], sem
            ).wait()
        pltpu.sync_copy(rows_v, out_hbm.at[pl.ds(base, per_w)])

    role_emb = _role_gather(roleset, r2)

    pos_tile = jnp.tile(pos_obj.reshape(_R, _D), (_BLK // _R, 1))
    obj_flat = pl.pallas_call(
        _obj_body,
        grid=(_ROWS // _BLK,),
        in_specs=[
            pl.BlockSpec((_BLK, _P), lambda i: (i, 0)),
            pl.BlockSpec((_BLK, 1), lambda i: (i, 0)),
            pl.BlockSpec((_NV, _P, _D), lambda i: (0, 0, 0)),
            pl.BlockSpec((1, _D), lambda i: (0, 0)),
            pl.BlockSpec((1, _D), lambda i: (0, 0)),
            pl.BlockSpec((1, _D), lambda i: (0, 0)),
            pl.BlockSpec((1, _D), lambda i: (0, 0)),
            pl.BlockSpec((_BLK, _D), lambda i: (0, 0)),
        ],
        out_specs=pl.BlockSpec((_BLK, _D), lambda i: (i, 0)),
        out_shape=jax.ShapeDtypeStruct((_ROWS, _D), f32),
        compiler_params=pltpu.CompilerParams(
            dimension_semantics=("parallel",)),
    )(props2, tr2, cw, b_t.reshape(1, _D), ln_gamma.reshape(1, _D),
      ln_beta.reshape(1, _D), mark_absent.reshape(1, _D), pos_tile)

    padding = thematic_roles == -1
    return (role_emb, obj_flat.reshape(_BS, _R, _D), padding)


# speedup vs baseline: 47.9501x; 1.0051x over previous
import functools

import jax
import jax.numpy as jnp
from jax import lax
from jax.experimental import pallas as pl
from jax.experimental.pallas import tpu as pltpu
from jax.experimental.pallas import tpu_sc as plsc

_BS = 16384
_R = 8
_P = 18
_D = 64
_NV = 4
_ROWS = _BS * _R
_BLK = 2048
_EPS = 1e-5


def _prep_body(rt_ref, wrl_ref, brl_ref, posr_ref, vt_ref, wt_ref,
               r2_ref, c_ref):
    r2_ref[...] = jnp.zeros_like(r2_ref)
    r2_ref[:, :_D] = (
        jnp.dot(rt_ref[...], wrl_ref[...], preferred_element_type=jnp.float32)
        + brl_ref[...] + posr_ref[...]
    )
    for p in range(_P):
        c_ref[pl.ds(_NV * p, _NV), :] = jnp.dot(
            vt_ref[pl.ds(_NV * p, _NV), :],
            wt_ref[pl.ds(_D * p, _D), :],
            preferred_element_type=jnp.float32,
        )


def _obj_body(props_ref, tr_ref, cw_ref, bt_ref, g_ref, b_ref, mark_ref,
              pos_ref, out_ref):
    props = props_ref[...]
    h = bt_ref[...]
    for w in range(_NV):
        oh = (props == w).astype(jnp.float32)
        h = h + jnp.dot(oh, cw_ref[w], preferred_element_type=jnp.float32)
    h = jnp.maximum(h, 0.0)
    mu = jnp.mean(h, axis=1, keepdims=True)
    var = jnp.mean((h - mu) * (h - mu), axis=1, keepdims=True)
    y = (h - mu) / jnp.sqrt(var + _EPS) * g_ref[...] + b_ref[...]
    pad = tr_ref[...] == -1
    out_ref[...] = jnp.where(pad, mark_ref[...], y) + pos_ref[...]


def kernel(roleset, properties, thematic_roles, roleset_table, W_rl, b_rl,
           value_table, W_t, b_t, ln_gamma, ln_beta, pos_role, pos_obj,
           mark_absent):
    f32 = jnp.float32
    roleset = roleset.astype(jnp.int32)
    props2 = properties.reshape(_ROWS, _P).astype(jnp.int32)
    tr2 = thematic_roles.reshape(_ROWS, 1).astype(jnp.int32)

    n_rs = roleset_table.shape[0]
    r2, c_tab = pl.pallas_call(
        _prep_body,
        out_shape=[
            jax.ShapeDtypeStruct((n_rs, 128), f32),
            jax.ShapeDtypeStruct((_NV * _P, _D), f32),
        ],
    )(roleset_table, W_rl, b_rl.reshape(1, _D), pos_role.reshape(1, _D),
      value_table, W_t)

    cw = c_tab.reshape(_P, _NV, _D).transpose(1, 0, 2)

    info = plsc.get_sparse_core_info()
    nc, ns = info.num_cores, info.num_subcores
    nw = nc * ns
    per_w = _BS // nw
    ch = 128
    n_ch = per_w // ch

    mesh = plsc.VectorSubcoreMesh(core_axis_name="c", subcore_axis_name="s")

    @functools.partial(
        pl.kernel,
        out_type=jax.ShapeDtypeStruct((_BS, 128), f32),
        mesh=mesh,
        scratch_types=[
            pltpu.VMEM((per_w,), jnp.int32),
            pltpu.VMEM((per_w, 128), f32),
            pltpu.SemaphoreType.DMA,
        ],
    )
    def _role_gather(idx_hbm, tab_hbm, out_hbm, idx_v, rows_v, sem):
        wid = lax.axis_index("s") * nc + lax.axis_index("c")
        base = wid * per_w
        pltpu.sync_copy(idx_hbm.at[pl.ds(base, per_w)], idx_v)
        cps = [
            pltpu.async_copy(
                tab_hbm.at[idx_v.at[pl.ds(j * ch, ch)]],
                rows_v.at[pl.ds(j * ch, ch)], sem)
            for j in range(n_ch)
        ]
        for cp in cps:
            cp.wait()
        pltpu.sync_copy(rows_v, out_hbm.at[pl.ds(base, per_w)])

    role_emb = _role_gather(roleset, r2)[:, :_D]

    pos_tile = jnp.tile(pos_obj.reshape(_R, _D), (_BLK // _R, 1))
    obj_flat = pl.pallas_call(
        _obj_body,
        grid=(_ROWS // _BLK,),
        in_specs=[
            pl.BlockSpec((_BLK, _P), lambda i: (i, 0)),
            pl.BlockSpec((_BLK, 1), lambda i: (i, 0)),
            pl.BlockSpec((_NV, _P, _D), lambda i: (0, 0, 0)),
            pl.BlockSpec((1, _D), lambda i: (0, 0)),
            pl.BlockSpec((1, _D), lambda i: (0, 0)),
            pl.BlockSpec((1, _D), lambda i: (0, 0)),
            pl.BlockSpec((1, _D), lambda i: (0, 0)),
            pl.BlockSpec((_BLK, _D), lambda i: (0, 0)),
        ],
        out_specs=pl.BlockSpec((_BLK, _D), lambda i: (i, 0)),
        out_shape=jax.ShapeDtypeStruct((_ROWS, _D), f32),
        compiler_params=pltpu.CompilerParams(
            dimension_semantics=("parallel",)),
    )(props2, tr2, cw, b_t.reshape(1, _D), ln_gamma.reshape(1, _D),
      ln_beta.reshape(1, _D), mark_absent.reshape(1, _D), pos_tile)

    padding = thematic_roles == -1
    return (role_emb, obj_flat.reshape(_BS, _R, _D), padding)
